# Initial kernel scaffold; baseline (speedup 1.0000x reference)
import jax
import jax.numpy as jnp
from jax.experimental import pallas as pl


def kernel(x, edge_index, edge_weight, edge_attr, batch, params):
    raise NotImplementedError("write your pallas kernel here")


# speedup vs baseline: 3.1111x; 3.1111x over previous
import functools

import jax
import jax.numpy as jnp
from jax import lax
from jax.experimental import pallas as pl
from jax.experimental.pallas import tpu as pltpu
from jax.experimental.pallas import tpu_sc as plsc

N = 10000
E = 320000
H = 128
G = 64
INV_CUTOFF = 1.0 / 5.0

NC = 2
NS = 16
NW = NC * NS
EW = E // NW
CH = 80
NCHUNK = EW // CH
NP = 10240
ROWS_PER_TILE = NP // NS

BN = 2000
BE = 4000
F32 = jnp.float32


def _silu(v):
    return v * jax.nn.sigmoid(v)


def _init_body(x_ref, emb_ref, wa_ref, wb_ref, h_ref, ha_ref, hb_ref):
    io = lax.broadcasted_iota(jnp.int32, (BN, 128), 1)
    oh = (io == x_ref[...]).astype(F32)
    h = jnp.dot(oh, emb_ref[...], preferred_element_type=F32)
    h_ref[...] = h
    ha_ref[...] = jnp.dot(h, wa_ref[...], preferred_element_type=F32)
    hb_ref[...] = jnp.dot(h, wb_ref[...], preferred_element_type=F32)


def _tc_init(x2, emb_p, wa, wb):
    return pl.pallas_call(
        _init_body,
        grid=(N // BN,),
        in_specs=[
            pl.BlockSpec((BN, 1), lambda i: (i, 0)),
            pl.BlockSpec((128, 128), lambda i: (0, 0)),
            pl.BlockSpec((128, 128), lambda i: (0, 0)),
            pl.BlockSpec((128, 128), lambda i: (0, 0)),
        ],
        out_specs=[
            pl.BlockSpec((BN, 128), lambda i: (i, 0)),
            pl.BlockSpec((BN, 128), lambda i: (i, 0)),
            pl.BlockSpec((BN, 128), lambda i: (i, 0)),
        ],
        out_shape=[
            jax.ShapeDtypeStruct((N, 128), F32),
            jax.ShapeDtypeStruct((N, 128), F32),
            jax.ShapeDtypeStruct((N, 128), F32),
        ],
    )(x2, emb_p, wa, wb)


def _gather_body(ha_hbm, hb_hbm, row2_hbm, col2_hbm, out_hbm,
                 rowb, colb, hab, hbb, ob, sem_a, sem_b):
    c = lax.axis_index("c")
    s = lax.axis_index("s")
    w = c * NS + s
    pltpu.sync_copy(row2_hbm.at[w], rowb)
    pltpu.sync_copy(col2_hbm.at[w], colb)

    def chunk(ci, carry):
        cpa = pltpu.async_copy(ha_hbm.at[rowb.at[ci]], hab, sem_a)
        cpb = pltpu.async_copy(hb_hbm.at[colb.at[ci]], hbb, sem_b)
        cpa.wait()
        cpb.wait()

        def rowloop(i, cc):
            for j in range(8):
                sl = pl.ds(j * 16, 16)
                ob[i, sl] = hab[i, sl] + hbb[i, sl]
            return cc

        lax.fori_loop(0, CH, rowloop, 0)
        pltpu.sync_copy(ob, out_hbm.at[pl.ds(w * EW + ci * CH, CH)])
        return carry

    lax.fori_loop(0, NCHUNK, chunk, 0)


def _sc_gather(ha, hb, row2, col2):
    mesh = plsc.VectorSubcoreMesh(
        core_axis_name="c", subcore_axis_name="s",
        num_cores=NC, num_subcores=NS)
    fn = functools.partial(
        pl.kernel,
        out_type=jax.ShapeDtypeStruct((E, 128), F32),
        mesh=mesh,
        scratch_types=[
            pltpu.VMEM((NCHUNK, CH), jnp.int32),
            pltpu.VMEM((NCHUNK, CH), jnp.int32),
            pltpu.VMEM((CH, 128), F32),
            pltpu.VMEM((CH, 128), F32),
            pltpu.VMEM((CH, 128), F32),
            pltpu.SemaphoreType.DMA,
            pltpu.SemaphoreType.DMA,
        ],
    )(_gather_body)
    return fn(ha, hb, row2, col2)


def _edge_body(gsum_ref, attr_ref, ew_ref, wc_ref, wd_ref, b1_ref,
               w2_ref, b2_ref, m_ref):
    d = ew_ref[...] * INV_CUTOFF
    t = (gsum_ref[...]
         + jnp.dot(attr_ref[...], wc_ref[...], preferred_element_type=F32)
         + d * wd_ref[...]
         + b1_ref[...])
    t = _silu(t)
    m_ref[...] = _silu(
        jnp.dot(t, w2_ref[...], preferred_element_type=F32) + b2_ref[...])


def _tc_edge(gsum, edge_attr, ew2, wc, wd, b1, w2, b2):
    return pl.pallas_call(
        _edge_body,
        grid=(E // BE,),
        in_specs=[
            pl.BlockSpec((BE, 128), lambda i: (i, 0)),
            pl.BlockSpec((BE, 128), lambda i: (i, 0)),
            pl.BlockSpec((BE, 1), lambda i: (i, 0)),
            pl.BlockSpec((128, 128), lambda i: (0, 0)),
            pl.BlockSpec((1, 128), lambda i: (0, 0)),
            pl.BlockSpec((1, 128), lambda i: (0, 0)),
            pl.BlockSpec((128, 128), lambda i: (0, 0)),
            pl.BlockSpec((1, 128), lambda i: (0, 0)),
        ],
        out_specs=pl.BlockSpec((BE, 128), lambda i: (i, 0)),
        out_shape=jax.ShapeDtypeStruct((E, 128), F32),
    )(gsum, edge_attr, ew2, wc, wd, b1, w2, b2)


def _scatter_body(m_hbm, row2_hbm, out_hbm, rowb, mbuf, zbuf, shag):
    c = lax.axis_index("c")
    s = lax.axis_index("s")
    w = c * NS + s

    def zrow(i, cc):
        for j in range(8):
            zbuf[i, pl.ds(j * 16, 16)] = jnp.zeros((16,), F32)
        return cc

    lax.fori_loop(0, 128, zrow, 0)
    for k in range(5):
        pltpu.sync_copy(zbuf, shag.at[pl.ds(s * ROWS_PER_TILE + k * 128, 128)])
    plsc.subcore_barrier()

    pltpu.sync_copy(row2_hbm.at[w], rowb)

    def chunk(ci, carry):
        pltpu.sync_copy(m_hbm.at[pl.ds(w * EW + ci * CH, CH)], mbuf)
        pltpu.sync_copy(mbuf, shag.at[rowb.at[ci]], add=True)
        return carry

    lax.fori_loop(0, NCHUNK, chunk, 0)
    plsc.subcore_barrier()
    pltpu.sync_copy(shag.at[pl.ds(s * ROWS_PER_TILE, ROWS_PER_TILE)],
                    out_hbm.at[c, pl.ds(s * ROWS_PER_TILE, ROWS_PER_TILE)])


def _sc_scatter(m, row2):
    mesh = plsc.VectorSubcoreMesh(
        core_axis_name="c", subcore_axis_name="s",
        num_cores=NC, num_subcores=NS)
    fn = functools.partial(
        pl.kernel,
        out_type=jax.ShapeDtypeStruct((NC, NP, 128), F32),
        mesh=mesh,
        scratch_types=[
            pltpu.VMEM((NCHUNK, CH), jnp.int32),
            pltpu.VMEM((CH, 128), F32),
            pltpu.VMEM((128, 128), F32),
            pltpu.VMEM_SHARED((NP, 128), F32),
        ],
    )(_scatter_body)
    return fn(m, row2)


def _node_body(h_ref, a0_ref, a1_ref, w1h_ref, w1a_ref, b1_ref, w2_ref,
               b2_ref, wa_ref, wb_ref, hn_ref, ha_ref, hb_ref):
    agg = a0_ref[0] + a1_ref[0]
    u = _silu(jnp.dot(h_ref[...], w1h_ref[...], preferred_element_type=F32)
              + jnp.dot(agg, w1a_ref[...], preferred_element_type=F32)
              + b1_ref[...])
    hn = h_ref[...] + jnp.dot(u, w2_ref[...],
                              preferred_element_type=F32) + b2_ref[...]
    hn_ref[...] = hn
    ha_ref[...] = jnp.dot(hn, wa_ref[...], preferred_element_type=F32)
    hb_ref[...] = jnp.dot(hn, wb_ref[...], preferred_element_type=F32)


def _tc_node(h, aggp, w1h, w1a, b1, w2, b2, wa, wb):
    return pl.pallas_call(
        _node_body,
        grid=(N // BN,),
        in_specs=[
            pl.BlockSpec((BN, 128), lambda i: (i, 0)),
            pl.BlockSpec((1, BN, 128), lambda i: (0, i, 0)),
            pl.BlockSpec((1, BN, 128), lambda i: (1, i, 0)),
            pl.BlockSpec((128, 128), lambda i: (0, 0)),
            pl.BlockSpec((128, 128), lambda i: (0, 0)),
            pl.BlockSpec((1, 128), lambda i: (0, 0)),
            pl.BlockSpec((128, 128), lambda i: (0, 0)),
            pl.BlockSpec((1, 128), lambda i: (0, 0)),
            pl.BlockSpec((128, 128), lambda i: (0, 0)),
            pl.BlockSpec((128, 128), lambda i: (0, 0)),
        ],
        out_specs=[
            pl.BlockSpec((BN, 128), lambda i: (i, 0)),
            pl.BlockSpec((BN, 128), lambda i: (i, 0)),
            pl.BlockSpec((BN, 128), lambda i: (i, 0)),
        ],
        out_shape=[
            jax.ShapeDtypeStruct((N, 128), F32),
            jax.ShapeDtypeStruct((N, 128), F32),
            jax.ShapeDtypeStruct((N, 128), F32),
        ],
    )(h, aggp, aggp, w1h, w1a, b1, w2, b2, wa, wb)


def _pool_body(h_ref, batch_ref, linw_ref, linb_ref, out_ref, sums, cnts):
    i = pl.program_id(0)

    @pl.when(i == 0)
    def _():
        sums[...] = jnp.zeros_like(sums)
        cnts[...] = jnp.zeros_like(cnts)

    io = lax.broadcasted_iota(jnp.int32, (BN, G), 1)
    oh = (io == batch_ref[...]).astype(F32)
    dn = (((0,), (0,)), ((), ()))
    sums[...] += lax.dot_general(oh, h_ref[...], dn,
                                 preferred_element_type=F32)
    cnts[...] += lax.dot_general(oh, jnp.ones((BN, 128), F32), dn,
                                 preferred_element_type=F32)

    @pl.when(i == pl.num_programs(0) - 1)
    def _():
        pooled = sums[...] / jnp.maximum(cnts[...], 1.0)
        out_ref[...] = (jnp.dot(jnp.maximum(pooled, 0.0), linw_ref[...],
                                preferred_element_type=F32) + linb_ref[...])


def _tc_pool(h, batch2, lin_w, lin_b):
    return pl.pallas_call(
        _pool_body,
        grid=(N // BN,),
        in_specs=[
            pl.BlockSpec((BN, 128), lambda i: (i, 0)),
            pl.BlockSpec((BN, 1), lambda i: (i, 0)),
            pl.BlockSpec((128, 128), lambda i: (0, 0)),
            pl.BlockSpec((1, 128), lambda i: (0, 0)),
        ],
        out_specs=pl.BlockSpec((G, 128), lambda i: (0, 0)),
        out_shape=jax.ShapeDtypeStruct((G, 128), F32),
        scratch_shapes=[
            pltpu.VMEM((G, 128), F32),
            pltpu.VMEM((G, 128), F32),
        ],
    )(h, batch2, lin_w, lin_b)


def kernel(x, edge_index, edge_weight, edge_attr, batch, params):
    x2 = x.astype(jnp.int32).reshape(N, 1)
    row = edge_index[0].astype(jnp.int32)
    col = edge_index[1].astype(jnp.int32)
    row2 = row.reshape(NW, NCHUNK, CH)
    col2 = col.reshape(NW, NCHUNK, CH)
    ew2 = edge_weight.astype(F32).reshape(E, 1)
    batch2 = batch.astype(jnp.int32).reshape(N, 1)

    emb_p = jnp.zeros((128, 128), F32).at[:100].set(params['emb'])
    lays = params['layers']
    wa = [lp['e_w1'][0:H] for lp in lays]
    wb = [lp['e_w1'][H:2 * H] for lp in lays]
    wc = [lp['e_w1'][2 * H:2 * H + 128] for lp in lays]
    wd = [lp['e_w1'][2 * H + 128:2 * H + 129] for lp in lays]
    b1 = [lp['e_b1'].reshape(1, H) for lp in lays]
    w2 = [lp['e_w2'] for lp in lays]
    b2 = [lp['e_b2'].reshape(1, H) for lp in lays]
    w1h = [lp['n_w1'][0:H] for lp in lays]
    w1a = [lp['n_w1'][H:2 * H] for lp in lays]
    nb1 = [lp['n_b1'].reshape(1, H) for lp in lays]
    nw2 = [lp['n_w2'] for lp in lays]
    nb2 = [lp['n_b2'].reshape(1, H) for lp in lays]

    h, ha, hb = _tc_init(x2, emb_p, wa[0], wb[0])
    zero_w = jnp.zeros((H, H), F32)
    for l in range(3):
        gsum = _sc_gather(ha, hb, row2, col2)
        m = _tc_edge(gsum, edge_attr, ew2, wc[l], wd[l], b1[l], w2[l], b2[l])
        aggp = _sc_scatter(m, row2)
        nwa = wa[l + 1] if l + 1 < 3 else zero_w
        nwb = wb[l + 1] if l + 1 < 3 else zero_w
        h, ha, hb = _tc_node(h, aggp, w1h[l], w1a[l], nb1[l], nw2[l],
                             nb2[l], nwa, nwb)
    return _tc_pool(h, batch2, params['lin_w'], params['lin_b'].reshape(1, H))


# speedup vs baseline: 3.7836x; 1.2161x over previous
import functools

import jax
import jax.numpy as jnp
from jax import lax
from jax.experimental import pallas as pl
from jax.experimental.pallas import tpu as pltpu
from jax.experimental.pallas import tpu_sc as plsc

N = 10000
E = 320000
H = 128
G = 64
INV_CUTOFF = 1.0 / 5.0

NC = 2
NS = 16
NW = NC * NS
EW = E // NW
CH = 40
NCHUNK = EW // CH
NP = 10240
ROWS_PER_TILE = NP // NS

BN = 2000
BE = 4000
F32 = jnp.float32


def _silu(v):
    return v * jax.nn.sigmoid(v)


def _init_body(x_ref, emb_ref, wa_ref, wb_ref, h_ref, ha_ref, hb_ref):
    io = lax.broadcasted_iota(jnp.int32, (BN, 128), 1)
    oh = (io == x_ref[...]).astype(F32)
    h = jnp.dot(oh, emb_ref[...], preferred_element_type=F32)
    h_ref[...] = h
    ha_ref[...] = jnp.dot(h, wa_ref[...], preferred_element_type=F32)
    hb_ref[...] = jnp.dot(h, wb_ref[...], preferred_element_type=F32)


def _tc_init(x2, emb_p, wa, wb):
    return pl.pallas_call(
        _init_body,
        grid=(N // BN,),
        in_specs=[
            pl.BlockSpec((BN, 1), lambda i: (i, 0)),
            pl.BlockSpec((128, 128), lambda i: (0, 0)),
            pl.BlockSpec((128, 128), lambda i: (0, 0)),
            pl.BlockSpec((128, 128), lambda i: (0, 0)),
        ],
        out_specs=[
            pl.BlockSpec((BN, 128), lambda i: (i, 0)),
            pl.BlockSpec((BN, 128), lambda i: (i, 0)),
            pl.BlockSpec((BN, 128), lambda i: (i, 0)),
        ],
        out_shape=[
            jax.ShapeDtypeStruct((N, 128), F32),
            jax.ShapeDtypeStruct((N, 128), F32),
            jax.ShapeDtypeStruct((N, 128), F32),
        ],
    )(x2, emb_p, wa, wb)


def _gather_body(ha_hbm, hb_hbm, row2_hbm, col2_hbm, out_hbm,
                 rowb, colb, hab0, hab1, hbb0, hbb1, ob,
                 sa0, sa1, sb0, sb1):
    c = lax.axis_index("c")
    s = lax.axis_index("s")
    w = c * NS + s
    pltpu.sync_copy(row2_hbm.at[w], rowb)
    pltpu.sync_copy(col2_hbm.at[w], colb)
    habs = (hab0, hab1)
    hbbs = (hbb0, hbb1)
    sas = (sa0, sa1)
    sbs = (sb0, sb1)
    for b in range(2):
        pltpu.async_copy(ha_hbm.at[rowb.at[b]], habs[b], sas[b])
        pltpu.async_copy(hb_hbm.at[colb.at[b]], hbbs[b], sbs[b])

    def body2(m2, cc):
        for b in range(2):
            ci = m2 * 2 + b
            pltpu.make_async_copy(
                ha_hbm.at[rowb.at[ci]], habs[b], sas[b]).wait()
            pltpu.make_async_copy(
                hb_hbm.at[colb.at[ci]], hbbs[b], sbs[b]).wait()

            def rowloop(i, c2, _b=b):
                for j in range(8):
                    sl = pl.ds(j * 16, 16)
                    ob[i, sl] = habs[_b][i, sl] + hbbs[_b][i, sl]
                return c2

            lax.fori_loop(0, CH, rowloop, 0)
            pltpu.sync_copy(ob, out_hbm.at[pl.ds(w * EW + ci * CH, CH)])

            @pl.when(ci + 2 < NCHUNK)
            def _(b=b, ci=ci):
                pltpu.async_copy(ha_hbm.at[rowb.at[ci + 2]], habs[b], sas[b])
                pltpu.async_copy(hb_hbm.at[colb.at[ci + 2]], hbbs[b], sbs[b])
        return cc

    lax.fori_loop(0, NCHUNK // 2, body2, 0)


def _sc_gather(ha, hb, row2, col2):
    mesh = plsc.VectorSubcoreMesh(
        core_axis_name="c", subcore_axis_name="s",
        num_cores=NC, num_subcores=NS)
    fn = functools.partial(
        pl.kernel,
        out_type=jax.ShapeDtypeStruct((E, 128), F32),
        mesh=mesh,
        scratch_types=[
            pltpu.VMEM((NCHUNK, CH), jnp.int32),
            pltpu.VMEM((NCHUNK, CH), jnp.int32),
            pltpu.VMEM((CH, 128), F32),
            pltpu.VMEM((CH, 128), F32),
            pltpu.VMEM((CH, 128), F32),
            pltpu.VMEM((CH, 128), F32),
            pltpu.VMEM((CH, 128), F32),
            pltpu.SemaphoreType.DMA,
            pltpu.SemaphoreType.DMA,
            pltpu.SemaphoreType.DMA,
            pltpu.SemaphoreType.DMA,
        ],
    )(_gather_body)
    return fn(ha, hb, row2, col2)


def _edge_body(gsum_ref, attr_ref, ew_ref, wc_ref, wd_ref, b1_ref,
               w2_ref, b2_ref, m_ref):
    d = ew_ref[...] * INV_CUTOFF
    t = (gsum_ref[...]
         + jnp.dot(attr_ref[...], wc_ref[...], preferred_element_type=F32)
         + d * wd_ref[...]
         + b1_ref[...])
    t = _silu(t)
    m_ref[...] = _silu(
        jnp.dot(t, w2_ref[...], preferred_element_type=F32) + b2_ref[...])


def _tc_edge(gsum, edge_attr, ew2, wc, wd, b1, w2, b2):
    return pl.pallas_call(
        _edge_body,
        grid=(E // BE,),
        in_specs=[
            pl.BlockSpec((BE, 128), lambda i: (i, 0)),
            pl.BlockSpec((BE, 128), lambda i: (i, 0)),
            pl.BlockSpec((BE, 1), lambda i: (i, 0)),
            pl.BlockSpec((128, 128), lambda i: (0, 0)),
            pl.BlockSpec((1, 128), lambda i: (0, 0)),
            pl.BlockSpec((1, 128), lambda i: (0, 0)),
            pl.BlockSpec((128, 128), lambda i: (0, 0)),
            pl.BlockSpec((1, 128), lambda i: (0, 0)),
        ],
        out_specs=pl.BlockSpec((BE, 128), lambda i: (i, 0)),
        out_shape=jax.ShapeDtypeStruct((E, 128), F32),
    )(gsum, edge_attr, ew2, wc, wd, b1, w2, b2)


def _scatter_body(m_hbm, row2_hbm, out_hbm, rowb, mb0, mb1, shag,
                  sm0, sm1):
    c = lax.axis_index("c")
    s = lax.axis_index("s")
    w = c * NS + s

    def zrow(i, cc):
        for j in range(8):
            mb0[i, pl.ds(j * 16, 16)] = jnp.zeros((16,), F32)
        return cc

    lax.fori_loop(0, CH, zrow, 0)
    for k in range(ROWS_PER_TILE // CH):
        pltpu.sync_copy(mb0, shag.at[pl.ds(s * ROWS_PER_TILE + k * CH, CH)])
    plsc.subcore_barrier()

    pltpu.sync_copy(row2_hbm.at[w], rowb)
    mbs = (mb0, mb1)
    sms = (sm0, sm1)
    for b in range(2):
        pltpu.async_copy(m_hbm.at[pl.ds(w * EW + b * CH, CH)], mbs[b], sms[b])

    def body2(m2, cc):
        for b in range(2):
            ci = m2 * 2 + b
            pltpu.make_async_copy(
                m_hbm.at[pl.ds(w * EW + ci * CH, CH)], mbs[b], sms[b]).wait()
            pltpu.sync_copy(mbs[b], shag.at[rowb.at[ci]], add=True)

            @pl.when(ci + 2 < NCHUNK)
            def _(b=b, ci=ci):
                pltpu.async_copy(
                    m_hbm.at[pl.ds(w * EW + (ci + 2) * CH, CH)],
                    mbs[b], sms[b])
        return cc

    lax.fori_loop(0, NCHUNK // 2, body2, 0)
    plsc.subcore_barrier()
    pltpu.sync_copy(shag.at[pl.ds(s * ROWS_PER_TILE, ROWS_PER_TILE)],
                    out_hbm.at[c, pl.ds(s * ROWS_PER_TILE, ROWS_PER_TILE)])


def _sc_scatter(m, row2):
    mesh = plsc.VectorSubcoreMesh(
        core_axis_name="c", subcore_axis_name="s",
        num_cores=NC, num_subcores=NS)
    fn = functools.partial(
        pl.kernel,
        out_type=jax.ShapeDtypeStruct((NC, NP, 128), F32),
        mesh=mesh,
        scratch_types=[
            pltpu.VMEM((NCHUNK, CH), jnp.int32),
            pltpu.VMEM((CH, 128), F32),
            pltpu.VMEM((CH, 128), F32),
            pltpu.VMEM_SHARED((NP, 128), F32),
            pltpu.SemaphoreType.DMA,
            pltpu.SemaphoreType.DMA,
        ],
    )(_scatter_body)
    return fn(m, row2)


def _node_body(h_ref, a0_ref, a1_ref, w1h_ref, w1a_ref, b1_ref, w2_ref,
               b2_ref, wa_ref, wb_ref, hn_ref, ha_ref, hb_ref):
    agg = a0_ref[0] + a1_ref[0]
    u = _silu(jnp.dot(h_ref[...], w1h_ref[...], preferred_element_type=F32)
              + jnp.dot(agg, w1a_ref[...], preferred_element_type=F32)
              + b1_ref[...])
    hn = h_ref[...] + jnp.dot(u, w2_ref[...],
                              preferred_element_type=F32) + b2_ref[...]
    hn_ref[...] = hn
    ha_ref[...] = jnp.dot(hn, wa_ref[...], preferred_element_type=F32)
    hb_ref[...] = jnp.dot(hn, wb_ref[...], preferred_element_type=F32)


def _tc_node(h, aggp, w1h, w1a, b1, w2, b2, wa, wb):
    return pl.pallas_call(
        _node_body,
        grid=(N // BN,),
        in_specs=[
            pl.BlockSpec((BN, 128), lambda i: (i, 0)),
            pl.BlockSpec((1, BN, 128), lambda i: (0, i, 0)),
            pl.BlockSpec((1, BN, 128), lambda i: (1, i, 0)),
            pl.BlockSpec((128, 128), lambda i: (0, 0)),
            pl.BlockSpec((128, 128), lambda i: (0, 0)),
            pl.BlockSpec((1, 128), lambda i: (0, 0)),
            pl.BlockSpec((128, 128), lambda i: (0, 0)),
            pl.BlockSpec((1, 128), lambda i: (0, 0)),
            pl.BlockSpec((128, 128), lambda i: (0, 0)),
            pl.BlockSpec((128, 128), lambda i: (0, 0)),
        ],
        out_specs=[
            pl.BlockSpec((BN, 128), lambda i: (i, 0)),
            pl.BlockSpec((BN, 128), lambda i: (i, 0)),
            pl.BlockSpec((BN, 128), lambda i: (i, 0)),
        ],
        out_shape=[
            jax.ShapeDtypeStruct((N, 128), F32),
            jax.ShapeDtypeStruct((N, 128), F32),
            jax.ShapeDtypeStruct((N, 128), F32),
        ],
    )(h, aggp, aggp, w1h, w1a, b1, w2, b2, wa, wb)


def _pool_body(h_ref, batch_ref, linw_ref, linb_ref, out_ref, sums, cnts):
    i = pl.program_id(0)

    @pl.when(i == 0)
    def _():
        sums[...] = jnp.zeros_like(sums)
        cnts[...] = jnp.zeros_like(cnts)

    io = lax.broadcasted_iota(jnp.int32, (BN, G), 1)
    oh = (io == batch_ref[...]).astype(F32)
    dn = (((0,), (0,)), ((), ()))
    sums[...] += lax.dot_general(oh, h_ref[...], dn,
                                 preferred_element_type=F32)
    cnts[...] += lax.dot_general(oh, jnp.ones((BN, 128), F32), dn,
                                 preferred_element_type=F32)

    @pl.when(i == pl.num_programs(0) - 1)
    def _():
        pooled = sums[...] / jnp.maximum(cnts[...], 1.0)
        out_ref[...] = (jnp.dot(jnp.maximum(pooled, 0.0), linw_ref[...],
                                preferred_element_type=F32) + linb_ref[...])


def _tc_pool(h, batch2, lin_w, lin_b):
    return pl.pallas_call(
        _pool_body,
        grid=(N // BN,),
        in_specs=[
            pl.BlockSpec((BN, 128), lambda i: (i, 0)),
            pl.BlockSpec((BN, 1), lambda i: (i, 0)),
            pl.BlockSpec((128, 128), lambda i: (0, 0)),
            pl.BlockSpec((1, 128), lambda i: (0, 0)),
        ],
        out_specs=pl.BlockSpec((G, 128), lambda i: (0, 0)),
        out_shape=jax.ShapeDtypeStruct((G, 128), F32),
        scratch_shapes=[
            pltpu.VMEM((G, 128), F32),
            pltpu.VMEM((G, 128), F32),
        ],
    )(h, batch2, lin_w, lin_b)


def kernel(x, edge_index, edge_weight, edge_attr, batch, params):
    x2 = x.astype(jnp.int32).reshape(N, 1)
    row = edge_index[0].astype(jnp.int32)
    col = edge_index[1].astype(jnp.int32)
    row2 = row.reshape(NW, NCHUNK, CH)
    col2 = col.reshape(NW, NCHUNK, CH)
    ew2 = edge_weight.astype(F32).reshape(E, 1)
    batch2 = batch.astype(jnp.int32).reshape(N, 1)

    emb_p = jnp.zeros((128, 128), F32).at[:100].set(params['emb'])
    lays = params['layers']
    wa = [lp['e_w1'][0:H] for lp in lays]
    wb = [lp['e_w1'][H:2 * H] for lp in lays]
    wc = [lp['e_w1'][2 * H:2 * H + 128] for lp in lays]
    wd = [lp['e_w1'][2 * H + 128:2 * H + 129] for lp in lays]
    b1 = [lp['e_b1'].reshape(1, H) for lp in lays]
    w2 = [lp['e_w2'] for lp in lays]
    b2 = [lp['e_b2'].reshape(1, H) for lp in lays]
    w1h = [lp['n_w1'][0:H] for lp in lays]
    w1a = [lp['n_w1'][H:2 * H] for lp in lays]
    nb1 = [lp['n_b1'].reshape(1, H) for lp in lays]
    nw2 = [lp['n_w2'] for lp in lays]
    nb2 = [lp['n_b2'].reshape(1, H) for lp in lays]

    h, ha, hb = _tc_init(x2, emb_p, wa[0], wb[0])
    zero_w = jnp.zeros((H, H), F32)
    for l in range(3):
        gsum = _sc_gather(ha, hb, row2, col2)
        m = _tc_edge(gsum, edge_attr, ew2, wc[l], wd[l], b1[l], w2[l], b2[l])
        aggp = _sc_scatter(m, row2)
        nwa = wa[l + 1] if l + 1 < 3 else zero_w
        nwb = wb[l + 1] if l + 1 < 3 else zero_w
        h, ha, hb = _tc_node(h, aggp, w1h[l], w1a[l], nb1[l], nw2[l],
                             nb2[l], nwa, nwb)
    return _tc_pool(h, batch2, params['lin_w'], params['lin_b'].reshape(1, H))


# speedup vs baseline: 3.8676x; 1.0222x over previous
import functools

import jax
import jax.numpy as jnp
from jax import lax
from jax.experimental import pallas as pl
from jax.experimental.pallas import tpu as pltpu
from jax.experimental.pallas import tpu_sc as plsc

N = 10000
E = 320000
H = 128
G = 64
INV_CUTOFF = 1.0 / 5.0

NC = 2
NS = 16
NW = NC * NS
EW = E // NW
CH = 40
NCHUNK = EW // CH
NP = 10240
ROWS_PER_TILE = NP // NS

BN = 2000
BE = 4000
F32 = jnp.float32


def _silu(v):
    return v * jax.nn.sigmoid(v)


def _init_body(x_ref, emb_ref, wa_ref, wb_ref, h_ref, ha_ref, hb_ref):
    io = lax.broadcasted_iota(jnp.int32, (BN, 128), 1)
    oh = (io == x_ref[...]).astype(F32)
    h = jnp.dot(oh, emb_ref[...], preferred_element_type=F32)
    h_ref[...] = h
    ha_ref[...] = jnp.dot(h, wa_ref[...], preferred_element_type=F32)
    hb_ref[...] = jnp.dot(h, wb_ref[...], preferred_element_type=F32)


def _tc_init(x2, emb_p, wa, wb):
    return pl.pallas_call(
        _init_body,
        grid=(N // BN,),
        in_specs=[
            pl.BlockSpec((BN, 1), lambda i: (i, 0)),
            pl.BlockSpec((128, 128), lambda i: (0, 0)),
            pl.BlockSpec((128, 128), lambda i: (0, 0)),
            pl.BlockSpec((128, 128), lambda i: (0, 0)),
        ],
        out_specs=[
            pl.BlockSpec((BN, 128), lambda i: (i, 0)),
            pl.BlockSpec((BN, 128), lambda i: (i, 0)),
            pl.BlockSpec((BN, 128), lambda i: (i, 0)),
        ],
        out_shape=[
            jax.ShapeDtypeStruct((N, 128), F32),
            jax.ShapeDtypeStruct((N, 128), F32),
            jax.ShapeDtypeStruct((N, 128), F32),
        ],
    )(x2, emb_p, wa, wb)


def _gather_body(ha_hbm, hb_hbm, row2_hbm, col2_hbm, out_hbm,
                 rowb, colb, hab0, hab1, hbb0, hbb1, ob0, ob1,
                 sa0, sa1, sb0, sb1, so0, so1):
    c = lax.axis_index("c")
    s = lax.axis_index("s")
    w = c * NS + s
    pltpu.sync_copy(row2_hbm.at[w], rowb)
    pltpu.sync_copy(col2_hbm.at[w], colb)
    habs = (hab0, hab1)
    hbbs = (hbb0, hbb1)
    obs = (ob0, ob1)
    sas = (sa0, sa1)
    sbs = (sb0, sb1)
    sos = (so0, so1)
    for b in range(2):
        pltpu.async_copy(ha_hbm.at[rowb.at[b]], habs[b], sas[b])
        pltpu.async_copy(hb_hbm.at[colb.at[b]], hbbs[b], sbs[b])

    def body2(m2, cc):
        for b in range(2):
            ci = m2 * 2 + b
            pltpu.make_async_copy(
                ha_hbm.at[rowb.at[ci]], habs[b], sas[b]).wait()
            pltpu.make_async_copy(
                hb_hbm.at[colb.at[ci]], hbbs[b], sbs[b]).wait()

            @pl.when(ci >= 2)
            def _(b=b):
                pltpu.make_async_copy(
                    obs[b], out_hbm.at[pl.ds(0, CH)], sos[b]).wait()

            def rowloop(i, c2, _b=b):
                for j in range(8):
                    sl = pl.ds(j * 16, 16)
                    obs[_b][i, sl] = habs[_b][i, sl] + hbbs[_b][i, sl]
                return c2

            lax.fori_loop(0, CH, rowloop, 0)
            pltpu.async_copy(
                obs[b], out_hbm.at[pl.ds(w * EW + ci * CH, CH)], sos[b])

            @pl.when(ci + 2 < NCHUNK)
            def _(b=b, ci=ci):
                pltpu.async_copy(ha_hbm.at[rowb.at[ci + 2]], habs[b], sas[b])
                pltpu.async_copy(hb_hbm.at[colb.at[ci + 2]], hbbs[b], sbs[b])
        return cc

    lax.fori_loop(0, NCHUNK // 2, body2, 0)
    for b in range(2):
        pltpu.make_async_copy(obs[b], out_hbm.at[pl.ds(0, CH)], sos[b]).wait()


def _sc_gather(ha, hb, row2, col2):
    mesh = plsc.VectorSubcoreMesh(
        core_axis_name="c", subcore_axis_name="s",
        num_cores=NC, num_subcores=NS)
    fn = functools.partial(
        pl.kernel,
        out_type=jax.ShapeDtypeStruct((E, 128), F32),
        mesh=mesh,
        scratch_types=[
            pltpu.VMEM((NCHUNK, CH), jnp.int32),
            pltpu.VMEM((NCHUNK, CH), jnp.int32),
            pltpu.VMEM((CH, 128), F32),
            pltpu.VMEM((CH, 128), F32),
            pltpu.VMEM((CH, 128), F32),
            pltpu.VMEM((CH, 128), F32),
            pltpu.VMEM((CH, 128), F32),
            pltpu.VMEM((CH, 128), F32),
            pltpu.SemaphoreType.DMA,
            pltpu.SemaphoreType.DMA,
            pltpu.SemaphoreType.DMA,
            pltpu.SemaphoreType.DMA,
            pltpu.SemaphoreType.DMA,
            pltpu.SemaphoreType.DMA,
        ],
    )(_gather_body)
    return fn(ha, hb, row2, col2)


def _edge_body(gsum_ref, attr_ref, ew_ref, wc_ref, wd_ref, b1_ref,
               w2_ref, b2_ref, m_ref):
    d = ew_ref[...] * INV_CUTOFF
    t = (gsum_ref[...]
         + jnp.dot(attr_ref[...], wc_ref[...].astype(jnp.bfloat16),
                   preferred_element_type=F32)
         + d * wd_ref[...]
         + b1_ref[...])
    t = _silu(t)
    m_ref[...] = _silu(
        jnp.dot(t, w2_ref[...], preferred_element_type=F32) + b2_ref[...])


def _tc_edge(gsum, edge_attr, ew2, wc, wd, b1, w2, b2):
    return pl.pallas_call(
        _edge_body,
        grid=(E // BE,),
        in_specs=[
            pl.BlockSpec((BE, 128), lambda i: (i, 0)),
            pl.BlockSpec((BE, 128), lambda i: (i, 0)),
            pl.BlockSpec((BE, 1), lambda i: (i, 0)),
            pl.BlockSpec((128, 128), lambda i: (0, 0)),
            pl.BlockSpec((1, 128), lambda i: (0, 0)),
            pl.BlockSpec((1, 128), lambda i: (0, 0)),
            pl.BlockSpec((128, 128), lambda i: (0, 0)),
            pl.BlockSpec((1, 128), lambda i: (0, 0)),
        ],
        out_specs=pl.BlockSpec((BE, 128), lambda i: (i, 0)),
        out_shape=jax.ShapeDtypeStruct((E, 128), F32),
    )(gsum, edge_attr, ew2, wc, wd, b1, w2, b2)


def _scatter_body(m_hbm, row2_hbm, out_hbm, rowb, mb0, mb1, shag,
                  sm0, sm1):
    c = lax.axis_index("c")
    s = lax.axis_index("s")
    w = c * NS + s

    def zrow(i, cc):
        for j in range(8):
            mb0[i, pl.ds(j * 16, 16)] = jnp.zeros((16,), F32)
        return cc

    lax.fori_loop(0, CH, zrow, 0)
    for k in range(ROWS_PER_TILE // CH):
        pltpu.sync_copy(mb0, shag.at[pl.ds(s * ROWS_PER_TILE + k * CH, CH)])
    plsc.subcore_barrier()

    pltpu.sync_copy(row2_hbm.at[w], rowb)
    mbs = (mb0, mb1)
    sms = (sm0, sm1)
    for b in range(2):
        pltpu.async_copy(m_hbm.at[pl.ds(w * EW + b * CH, CH)], mbs[b], sms[b])

    def body2(m2, cc):
        for b in range(2):
            ci = m2 * 2 + b
            pltpu.make_async_copy(
                m_hbm.at[pl.ds(w * EW + ci * CH, CH)], mbs[b], sms[b]).wait()
            pltpu.sync_copy(mbs[b], shag.at[rowb.at[ci]], add=True)

            @pl.when(ci + 2 < NCHUNK)
            def _(b=b, ci=ci):
                pltpu.async_copy(
                    m_hbm.at[pl.ds(w * EW + (ci + 2) * CH, CH)],
                    mbs[b], sms[b])
        return cc

    lax.fori_loop(0, NCHUNK // 2, body2, 0)
    plsc.subcore_barrier()
    pltpu.sync_copy(shag.at[pl.ds(s * ROWS_PER_TILE, ROWS_PER_TILE)],
                    out_hbm.at[c, pl.ds(s * ROWS_PER_TILE, ROWS_PER_TILE)])


def _sc_scatter(m, row2):
    mesh = plsc.VectorSubcoreMesh(
        core_axis_name="c", subcore_axis_name="s",
        num_cores=NC, num_subcores=NS)
    fn = functools.partial(
        pl.kernel,
        out_type=jax.ShapeDtypeStruct((NC, NP, 128), F32),
        mesh=mesh,
        scratch_types=[
            pltpu.VMEM((NCHUNK, CH), jnp.int32),
            pltpu.VMEM((CH, 128), F32),
            pltpu.VMEM((CH, 128), F32),
            pltpu.VMEM_SHARED((NP, 128), F32),
            pltpu.SemaphoreType.DMA,
            pltpu.SemaphoreType.DMA,
        ],
    )(_scatter_body)
    return fn(m, row2)


def _node_body(h_ref, a0_ref, a1_ref, w1h_ref, w1a_ref, b1_ref, w2_ref,
               b2_ref, wa_ref, wb_ref, hn_ref, ha_ref, hb_ref):
    agg = a0_ref[0] + a1_ref[0]
    u = _silu(jnp.dot(h_ref[...], w1h_ref[...], preferred_element_type=F32)
              + jnp.dot(agg, w1a_ref[...], preferred_element_type=F32)
              + b1_ref[...])
    hn = h_ref[...] + jnp.dot(u, w2_ref[...],
                              preferred_element_type=F32) + b2_ref[...]
    hn_ref[...] = hn
    ha_ref[...] = jnp.dot(hn, wa_ref[...], preferred_element_type=F32)
    hb_ref[...] = jnp.dot(hn, wb_ref[...], preferred_element_type=F32)


def _tc_node(h, aggp, w1h, w1a, b1, w2, b2, wa, wb):
    return pl.pallas_call(
        _node_body,
        grid=(N // BN,),
        in_specs=[
            pl.BlockSpec((BN, 128), lambda i: (i, 0)),
            pl.BlockSpec((1, BN, 128), lambda i: (0, i, 0)),
            pl.BlockSpec((1, BN, 128), lambda i: (1, i, 0)),
            pl.BlockSpec((128, 128), lambda i: (0, 0)),
            pl.BlockSpec((128, 128), lambda i: (0, 0)),
            pl.BlockSpec((1, 128), lambda i: (0, 0)),
            pl.BlockSpec((128, 128), lambda i: (0, 0)),
            pl.BlockSpec((1, 128), lambda i: (0, 0)),
            pl.BlockSpec((128, 128), lambda i: (0, 0)),
            pl.BlockSpec((128, 128), lambda i: (0, 0)),
        ],
        out_specs=[
            pl.BlockSpec((BN, 128), lambda i: (i, 0)),
            pl.BlockSpec((BN, 128), lambda i: (i, 0)),
            pl.BlockSpec((BN, 128), lambda i: (i, 0)),
        ],
        out_shape=[
            jax.ShapeDtypeStruct((N, 128), F32),
            jax.ShapeDtypeStruct((N, 128), F32),
            jax.ShapeDtypeStruct((N, 128), F32),
        ],
    )(h, aggp, aggp, w1h, w1a, b1, w2, b2, wa, wb)


def _pool_body(h_ref, batch_ref, linw_ref, linb_ref, out_ref, sums, cnts):
    i = pl.program_id(0)

    @pl.when(i == 0)
    def _():
        sums[...] = jnp.zeros_like(sums)
        cnts[...] = jnp.zeros_like(cnts)

    io = lax.broadcasted_iota(jnp.int32, (BN, G), 1)
    oh = (io == batch_ref[...]).astype(F32)
    dn = (((0,), (0,)), ((), ()))
    sums[...] += lax.dot_general(oh, h_ref[...], dn,
                                 preferred_element_type=F32)
    cnts[...] += lax.dot_general(oh, jnp.ones((BN, 128), F32), dn,
                                 preferred_element_type=F32)

    @pl.when(i == pl.num_programs(0) - 1)
    def _():
        pooled = sums[...] / jnp.maximum(cnts[...], 1.0)
        out_ref[...] = (jnp.dot(jnp.maximum(pooled, 0.0), linw_ref[...],
                                preferred_element_type=F32) + linb_ref[...])


def _tc_pool(h, batch2, lin_w, lin_b):
    return pl.pallas_call(
        _pool_body,
        grid=(N // BN,),
        in_specs=[
            pl.BlockSpec((BN, 128), lambda i: (i, 0)),
            pl.BlockSpec((BN, 1), lambda i: (i, 0)),
            pl.BlockSpec((128, 128), lambda i: (0, 0)),
            pl.BlockSpec((1, 128), lambda i: (0, 0)),
        ],
        out_specs=pl.BlockSpec((G, 128), lambda i: (0, 0)),
        out_shape=jax.ShapeDtypeStruct((G, 128), F32),
        scratch_shapes=[
            pltpu.VMEM((G, 128), F32),
            pltpu.VMEM((G, 128), F32),
        ],
    )(h, batch2, lin_w, lin_b)


def kernel(x, edge_index, edge_weight, edge_attr, batch, params):
    x2 = x.astype(jnp.int32).reshape(N, 1)
    row = edge_index[0].astype(jnp.int32)
    col = edge_index[1].astype(jnp.int32)
    row2 = row.reshape(NW, NCHUNK, CH)
    col2 = col.reshape(NW, NCHUNK, CH)
    ew2 = edge_weight.astype(F32).reshape(E, 1)
    batch2 = batch.astype(jnp.int32).reshape(N, 1)

    emb_p = jnp.zeros((128, 128), F32).at[:100].set(params['emb'])
    lays = params['layers']
    wa = [lp['e_w1'][0:H] for lp in lays]
    wb = [lp['e_w1'][H:2 * H] for lp in lays]
    wc = [lp['e_w1'][2 * H:2 * H + 128] for lp in lays]
    wd = [lp['e_w1'][2 * H + 128:2 * H + 129] for lp in lays]
    b1 = [lp['e_b1'].reshape(1, H) for lp in lays]
    w2 = [lp['e_w2'] for lp in lays]
    b2 = [lp['e_b2'].reshape(1, H) for lp in lays]
    w1h = [lp['n_w1'][0:H] for lp in lays]
    w1a = [lp['n_w1'][H:2 * H] for lp in lays]
    nb1 = [lp['n_b1'].reshape(1, H) for lp in lays]
    nw2 = [lp['n_w2'] for lp in lays]
    nb2 = [lp['n_b2'].reshape(1, H) for lp in lays]

    h, ha, hb = _tc_init(x2, emb_p, wa[0], wb[0])
    zero_w = jnp.zeros((H, H), F32)
    attr_bf = edge_attr.astype(jnp.bfloat16)
    for l in range(3):
        gsum = _sc_gather(ha, hb, row2, col2)
        m = _tc_edge(gsum, attr_bf, ew2, wc[l], wd[l], b1[l], w2[l], b2[l])
        aggp = _sc_scatter(m, row2)
        nwa = wa[l + 1] if l + 1 < 3 else zero_w
        nwb = wb[l + 1] if l + 1 < 3 else zero_w
        h, ha, hb = _tc_node(h, aggp, w1h[l], w1a[l], nb1[l], nw2[l],
                             nb2[l], nwa, nwb)
    return _tc_pool(h, batch2, params['lin_w'], params['lin_b'].reshape(1, H))


# speedup vs baseline: 4.0003x; 1.0343x over previous
import functools

import jax
import jax.numpy as jnp
from jax import lax
from jax.experimental import pallas as pl
from jax.experimental.pallas import tpu as pltpu
from jax.experimental.pallas import tpu_sc as plsc

N = 10000
E = 320000
H = 128
G = 64
INV_CUTOFF = 1.0 / 5.0

NC = 2
NS = 16
NW = NC * NS
EW = E // NW
CH = 40
NCHUNK = EW // CH
NHALF = 2
E_H = E // NHALF
EW_H = EW // NHALF
NCHUNK_H = NCHUNK // NHALF
NP = 10240
ROWS_PER_TILE = NP // NS

BN = 2000
BE = 4000
F32 = jnp.float32
BF16 = jnp.bfloat16


def _silu(v):
    return v * jax.nn.sigmoid(v)


def _sc_mesh():
    return plsc.VectorSubcoreMesh(
        core_axis_name="c", subcore_axis_name="s",
        num_cores=NC, num_subcores=NS)


def _init_body(x_ref, emb_ref, wa_ref, wb_ref, h_ref, ha_ref, hb_ref):
    io = lax.broadcasted_iota(jnp.int32, (BN, 128), 1)
    oh = (io == x_ref[...]).astype(F32)
    h = jnp.dot(oh, emb_ref[...], preferred_element_type=F32)
    h_ref[...] = h
    ha_ref[...] = jnp.dot(h, wa_ref[...], preferred_element_type=F32)
    hb_ref[...] = jnp.dot(h, wb_ref[...], preferred_element_type=F32)


def _tc_init(x2, emb_p, wa, wb):
    return pl.pallas_call(
        _init_body,
        grid=(N // BN,),
        in_specs=[
            pl.BlockSpec((BN, 1), lambda i: (i, 0)),
            pl.BlockSpec((128, 128), lambda i: (0, 0)),
            pl.BlockSpec((128, 128), lambda i: (0, 0)),
            pl.BlockSpec((128, 128), lambda i: (0, 0)),
        ],
        out_specs=[
            pl.BlockSpec((BN, 128), lambda i: (i, 0)),
            pl.BlockSpec((BN, 128), lambda i: (i, 0)),
            pl.BlockSpec((BN, 128), lambda i: (i, 0)),
        ],
        out_shape=[
            jax.ShapeDtypeStruct((N, 128), F32),
            jax.ShapeDtypeStruct((N, 128), F32),
            jax.ShapeDtypeStruct((N, 128), F32),
        ],
    )(x2, emb_p, wa, wb)


def _make_gather_body(ew, nchunk):
    def body(ha_hbm, hb_hbm, row2_hbm, col2_hbm, out_hbm,
             rowb, colb, hab0, hab1, hbb0, hbb1, ob0, ob1,
             sa0, sa1, sb0, sb1, so0, so1):
        c = lax.axis_index("c")
        s = lax.axis_index("s")
        w = c * NS + s
        pltpu.sync_copy(row2_hbm.at[w], rowb)
        pltpu.sync_copy(col2_hbm.at[w], colb)
        habs = (hab0, hab1)
        hbbs = (hbb0, hbb1)
        obs = (ob0, ob1)
        sas = (sa0, sa1)
        sbs = (sb0, sb1)
        sos = (so0, so1)
        for b in range(2):
            pltpu.async_copy(ha_hbm.at[rowb.at[b]], habs[b], sas[b])
            pltpu.async_copy(hb_hbm.at[colb.at[b]], hbbs[b], sbs[b])

        def process(b, ci, first_reuse):
            pltpu.make_async_copy(
                ha_hbm.at[rowb.at[ci]], habs[b], sas[b]).wait()
            pltpu.make_async_copy(
                hb_hbm.at[colb.at[ci]], hbbs[b], sbs[b]).wait()

            @pl.when(first_reuse)
            def _():
                pltpu.make_async_copy(
                    obs[b], out_hbm.at[pl.ds(0, CH)], sos[b]).wait()

            def rowloop(i, c2):
                for j in range(8):
                    sl = pl.ds(j * 16, 16)
                    obs[b][i, sl] = habs[b][i, sl] + hbbs[b][i, sl]
                return c2

            lax.fori_loop(0, CH, rowloop, 0)
            pltpu.async_copy(
                obs[b], out_hbm.at[pl.ds(w * ew + ci * CH, CH)], sos[b])

            @pl.when(ci + 2 < nchunk)
            def _():
                pltpu.async_copy(ha_hbm.at[rowb.at[ci + 2]], habs[b], sas[b])
                pltpu.async_copy(hb_hbm.at[colb.at[ci + 2]], hbbs[b], sbs[b])

        def body2(m2, cc):
            for b in range(2):
                ci = m2 * 2 + b
                process(b, ci, ci >= 2)
            return cc

        lax.fori_loop(0, nchunk // 2, body2, 0)
        if nchunk % 2:
            process(0, nchunk - 1, nchunk - 1 >= 2)
        for b in range(2):
            pltpu.make_async_copy(
                obs[b], out_hbm.at[pl.ds(0, CH)], sos[b]).wait()

    return body


def _sc_gather(ha, hb, row2, col2, e_sz, ew, nchunk):
    fn = functools.partial(
        pl.kernel,
        out_type=jax.ShapeDtypeStruct((e_sz, 128), F32),
        mesh=_sc_mesh(),
        scratch_types=[
            pltpu.VMEM((nchunk, CH), jnp.int32),
            pltpu.VMEM((nchunk, CH), jnp.int32),
            pltpu.VMEM((CH, 128), F32),
            pltpu.VMEM((CH, 128), F32),
            pltpu.VMEM((CH, 128), F32),
            pltpu.VMEM((CH, 128), F32),
            pltpu.VMEM((CH, 128), F32),
            pltpu.VMEM((CH, 128), F32),
            pltpu.SemaphoreType.DMA,
            pltpu.SemaphoreType.DMA,
            pltpu.SemaphoreType.DMA,
            pltpu.SemaphoreType.DMA,
            pltpu.SemaphoreType.DMA,
            pltpu.SemaphoreType.DMA,
        ],
    )(_make_gather_body(ew, nchunk))
    return fn(ha, hb, row2, col2)


def _edge_body(gsum_ref, attr_ref, ew_ref, wc_ref, wd_ref, b1_ref,
               w2_ref, b2_ref, m_ref):
    d = ew_ref[...] * INV_CUTOFF
    t = (gsum_ref[...]
         + jnp.dot(attr_ref[...], wc_ref[...].astype(BF16),
                   preferred_element_type=F32)
         + d * wd_ref[...]
         + b1_ref[...])
    t = _silu(t)
    m_ref[...] = _silu(
        jnp.dot(t, w2_ref[...], preferred_element_type=F32) + b2_ref[...])


def _tc_edge(gsum, edge_attr, ew2, wc, wd, b1, w2, b2, e_sz):
    return pl.pallas_call(
        _edge_body,
        grid=(e_sz // BE,),
        in_specs=[
            pl.BlockSpec((BE, 128), lambda i: (i, 0)),
            pl.BlockSpec((BE, 128), lambda i: (i, 0)),
            pl.BlockSpec((BE, 1), lambda i: (i, 0)),
            pl.BlockSpec((128, 128), lambda i: (0, 0)),
            pl.BlockSpec((1, 128), lambda i: (0, 0)),
            pl.BlockSpec((1, 128), lambda i: (0, 0)),
            pl.BlockSpec((128, 128), lambda i: (0, 0)),
            pl.BlockSpec((1, 128), lambda i: (0, 0)),
        ],
        out_specs=pl.BlockSpec((BE, 128), lambda i: (i, 0)),
        out_shape=jax.ShapeDtypeStruct((e_sz, 128), F32),
    )(gsum, edge_attr, ew2, wc, wd, b1, w2, b2)


def _make_scatter_body(ew, nchunk):
    def body(m_hbm, row2_hbm, out_hbm, rowb, mb0, mb1, shag, sm0, sm1):
        c = lax.axis_index("c")
        s = lax.axis_index("s")
        w = c * NS + s

        def zrow(i, cc):
            for j in range(8):
                mb0[i, pl.ds(j * 16, 16)] = jnp.zeros((16,), F32)
            return cc

        lax.fori_loop(0, CH, zrow, 0)
        for k in range(ROWS_PER_TILE // CH):
            pltpu.sync_copy(mb0,
                            shag.at[pl.ds(s * ROWS_PER_TILE + k * CH, CH)])
        plsc.subcore_barrier()

        pltpu.sync_copy(row2_hbm.at[w], rowb)
        mbs = (mb0, mb1)
        sms = (sm0, sm1)
        for b in range(2):
            pltpu.async_copy(
                m_hbm.at[pl.ds(w * ew + b * CH, CH)], mbs[b], sms[b])

        def process(b, ci):
            pltpu.make_async_copy(
                m_hbm.at[pl.ds(w * ew + ci * CH, CH)], mbs[b], sms[b]).wait()
            pltpu.sync_copy(mbs[b], shag.at[rowb.at[ci]], add=True)

            @pl.when(ci + 2 < nchunk)
            def _():
                pltpu.async_copy(
                    m_hbm.at[pl.ds(w * ew + (ci + 2) * CH, CH)],
                    mbs[b], sms[b])

        def body2(m2, cc):
            for b in range(2):
                process(b, m2 * 2 + b)
            return cc

        lax.fori_loop(0, nchunk // 2, body2, 0)
        if nchunk % 2:
            process(0, nchunk - 1)
        plsc.subcore_barrier()
        pltpu.sync_copy(shag.at[pl.ds(s * ROWS_PER_TILE, ROWS_PER_TILE)],
                        out_hbm.at[c, pl.ds(s * ROWS_PER_TILE,
                                            ROWS_PER_TILE)])

    return body


def _sc_scatter(m, row2, ew, nchunk):
    fn = functools.partial(
        pl.kernel,
        out_type=jax.ShapeDtypeStruct((NC, NP, 128), F32),
        mesh=_sc_mesh(),
        scratch_types=[
            pltpu.VMEM((nchunk, CH), jnp.int32),
            pltpu.VMEM((CH, 128), F32),
            pltpu.VMEM((CH, 128), F32),
            pltpu.VMEM_SHARED((NP, 128), F32),
            pltpu.SemaphoreType.DMA,
            pltpu.SemaphoreType.DMA,
        ],
    )(_make_scatter_body(ew, nchunk))
    return fn(m, row2)


def _node_body(h_ref, a0_ref, a1_ref, a2_ref, a3_ref, w1h_ref, w1a_ref,
               b1_ref, w2_ref, b2_ref, wa_ref, wb_ref,
               hn_ref, ha_ref, hb_ref):
    agg = (a0_ref[0] + a1_ref[0]) + (a2_ref[0] + a3_ref[0])
    u = _silu(jnp.dot(h_ref[...], w1h_ref[...], preferred_element_type=F32)
              + jnp.dot(agg, w1a_ref[...], preferred_element_type=F32)
              + b1_ref[...])
    hn = h_ref[...] + jnp.dot(u, w2_ref[...],
                              preferred_element_type=F32) + b2_ref[...]
    hn_ref[...] = hn
    ha_ref[...] = jnp.dot(hn, wa_ref[...], preferred_element_type=F32)
    hb_ref[...] = jnp.dot(hn, wb_ref[...], preferred_element_type=F32)


def _tc_node(h, aggpA, aggpB, w1h, w1a, b1, w2, b2, wa, wb):
    wspec = pl.BlockSpec((128, 128), lambda i: (0, 0))
    bspec = pl.BlockSpec((1, 128), lambda i: (0, 0))
    return pl.pallas_call(
        _node_body,
        grid=(N // BN,),
        in_specs=[
            pl.BlockSpec((BN, 128), lambda i: (i, 0)),
            pl.BlockSpec((1, BN, 128), lambda i: (0, i, 0)),
            pl.BlockSpec((1, BN, 128), lambda i: (1, i, 0)),
            pl.BlockSpec((1, BN, 128), lambda i: (0, i, 0)),
            pl.BlockSpec((1, BN, 128), lambda i: (1, i, 0)),
            wspec, wspec, bspec, wspec, bspec, wspec, wspec,
        ],
        out_specs=[
            pl.BlockSpec((BN, 128), lambda i: (i, 0)),
            pl.BlockSpec((BN, 128), lambda i: (i, 0)),
            pl.BlockSpec((BN, 128), lambda i: (i, 0)),
        ],
        out_shape=[
            jax.ShapeDtypeStruct((N, 128), F32),
            jax.ShapeDtypeStruct((N, 128), F32),
            jax.ShapeDtypeStruct((N, 128), F32),
        ],
    )(h, aggpA, aggpA, aggpB, aggpB, w1h, w1a, b1, w2, b2, wa, wb)


def _pool_body(h_ref, batch_ref, linw_ref, linb_ref, out_ref, sums, cnts):
    i = pl.program_id(0)

    @pl.when(i == 0)
    def _():
        sums[...] = jnp.zeros_like(sums)
        cnts[...] = jnp.zeros_like(cnts)

    io = lax.broadcasted_iota(jnp.int32, (BN, G), 1)
    oh = (io == batch_ref[...]).astype(F32)
    dn = (((0,), (0,)), ((), ()))
    sums[...] += lax.dot_general(oh, h_ref[...], dn,
                                 preferred_element_type=F32)
    cnts[...] += lax.dot_general(oh, jnp.ones((BN, 128), F32), dn,
                                 preferred_element_type=F32)

    @pl.when(i == pl.num_programs(0) - 1)
    def _():
        pooled = sums[...] / jnp.maximum(cnts[...], 1.0)
        out_ref[...] = (jnp.dot(jnp.maximum(pooled, 0.0), linw_ref[...],
                                preferred_element_type=F32) + linb_ref[...])


def _tc_pool(h, batch2, lin_w, lin_b):
    return pl.pallas_call(
        _pool_body,
        grid=(N // BN,),
        in_specs=[
            pl.BlockSpec((BN, 128), lambda i: (i, 0)),
            pl.BlockSpec((BN, 1), lambda i: (i, 0)),
            pl.BlockSpec((128, 128), lambda i: (0, 0)),
            pl.BlockSpec((1, 128), lambda i: (0, 0)),
        ],
        out_specs=pl.BlockSpec((G, 128), lambda i: (0, 0)),
        out_shape=jax.ShapeDtypeStruct((G, 128), F32),
        scratch_shapes=[
            pltpu.VMEM((G, 128), F32),
            pltpu.VMEM((G, 128), F32),
        ],
    )(h, batch2, lin_w, lin_b)


def kernel(x, edge_index, edge_weight, edge_attr, batch, params):
    x2 = x.astype(jnp.int32).reshape(N, 1)
    row = edge_index[0].astype(jnp.int32)
    col = edge_index[1].astype(jnp.int32)
    row3 = row.reshape(NW, NCHUNK, CH)
    col3 = col.reshape(NW, NCHUNK, CH)
    rowA, rowB = row3[:, :NCHUNK_H], row3[:, NCHUNK_H:]
    colA, colB = col3[:, :NCHUNK_H], col3[:, NCHUNK_H:]
    batch2 = batch.astype(jnp.int32).reshape(N, 1)

    attr_bf = edge_attr.astype(BF16).reshape(NW, NHALF, EW_H, 128)
    attrA = attr_bf[:, 0].reshape(E_H, 128)
    attrB = attr_bf[:, 1].reshape(E_H, 128)
    ew4 = edge_weight.astype(F32).reshape(NW, NHALF, EW_H, 1)
    ewA = ew4[:, 0].reshape(E_H, 1)
    ewB = ew4[:, 1].reshape(E_H, 1)

    emb_p = jnp.zeros((128, 128), F32).at[:100].set(params['emb'])
    lays = params['layers']
    wa = [lp['e_w1'][0:H] for lp in lays]
    wb = [lp['e_w1'][H:2 * H] for lp in lays]
    wc = [lp['e_w1'][2 * H:2 * H + 128] for lp in lays]
    wd = [lp['e_w1'][2 * H + 128:2 * H + 129] for lp in lays]
    b1 = [lp['e_b1'].reshape(1, H) for lp in lays]
    w2 = [lp['e_w2'] for lp in lays]
    b2 = [lp['e_b2'].reshape(1, H) for lp in lays]
    w1h = [lp['n_w1'][0:H] for lp in lays]
    w1a = [lp['n_w1'][H:2 * H] for lp in lays]
    nb1 = [lp['n_b1'].reshape(1, H) for lp in lays]
    nw2 = [lp['n_w2'] for lp in lays]
    nb2 = [lp['n_b2'].reshape(1, H) for lp in lays]

    h, ha, hb = _tc_init(x2, emb_p, wa[0], wb[0])
    zero_w = jnp.zeros((H, H), F32)
    for l in range(3):
        gA = _sc_gather(ha, hb, rowA, colA, E_H, EW_H, NCHUNK_H)
        mA = _tc_edge(gA, attrA, ewA, wc[l], wd[l], b1[l], w2[l], b2[l], E_H)
        gB = _sc_gather(ha, hb, rowB, colB, E_H, EW_H, NCHUNK_H)
        mB = _tc_edge(gB, attrB, ewB, wc[l], wd[l], b1[l], w2[l], b2[l], E_H)
        aggpA = _sc_scatter(mA, rowA, EW_H, NCHUNK_H)
        aggpB = _sc_scatter(mB, rowB, EW_H, NCHUNK_H)
        nwa = wa[l + 1] if l + 1 < 3 else zero_w
        nwb = wb[l + 1] if l + 1 < 3 else zero_w
        h, ha, hb = _tc_node(h, aggpA, aggpB, w1h[l], w1a[l], nb1[l],
                             nw2[l], nb2[l], nwa, nwb)
    return _tc_pool(h, batch2, params['lin_w'], params['lin_b'].reshape(1, H))


# speedup vs baseline: 4.2102x; 1.0525x over previous
import functools

import jax
import jax.numpy as jnp
from jax import lax
from jax.experimental import pallas as pl
from jax.experimental.pallas import tpu as pltpu
from jax.experimental.pallas import tpu_sc as plsc

N = 10000
E = 320000
H = 128
G = 64
INV_CUTOFF = 1.0 / 5.0

NC = 2
NS = 16
NW = NC * NS
EW = E // NW
CH = 40
NCHUNK = EW // CH
NHALF = 2
E_H = E // NHALF
EW_H = EW // NHALF
NCHUNK_H = NCHUNK // NHALF
NP = 10240
ROWS_PER_TILE = NP // NS

BN = 2000
BE = 4000
F32 = jnp.float32
BF16 = jnp.bfloat16


def _silu(v):
    return v * jax.nn.sigmoid(v)


def _sc_mesh():
    return plsc.VectorSubcoreMesh(
        core_axis_name="c", subcore_axis_name="s",
        num_cores=NC, num_subcores=NS)


def _init_body(x_ref, emb_ref, wa_ref, wb_ref, h_ref, ha_ref, hb_ref):
    io = lax.broadcasted_iota(jnp.int32, (BN, 128), 1)
    oh = (io == x_ref[...]).astype(F32)
    h = jnp.dot(oh, emb_ref[...], preferred_element_type=F32)
    h_ref[...] = h
    ha_ref[...] = jnp.dot(h, wa_ref[...], preferred_element_type=F32)
    hb_ref[...] = jnp.dot(h, wb_ref[...], preferred_element_type=F32)


def _tc_init(x2, emb_p, wa, wb):
    return pl.pallas_call(
        _init_body,
        grid=(N // BN,),
        in_specs=[
            pl.BlockSpec((BN, 1), lambda i: (i, 0)),
            pl.BlockSpec((128, 128), lambda i: (0, 0)),
            pl.BlockSpec((128, 128), lambda i: (0, 0)),
            pl.BlockSpec((128, 128), lambda i: (0, 0)),
        ],
        out_specs=[
            pl.BlockSpec((BN, 128), lambda i: (i, 0)),
            pl.BlockSpec((BN, 128), lambda i: (i, 0)),
            pl.BlockSpec((BN, 128), lambda i: (i, 0)),
        ],
        out_shape=[
            jax.ShapeDtypeStruct((N, 128), F32),
            jax.ShapeDtypeStruct((N, 128), F32),
            jax.ShapeDtypeStruct((N, 128), F32),
        ],
    )(x2, emb_p, wa, wb)


def _make_gather_body(ew, nchunk):
    def body(ha_hbm, hb_hbm, row2_hbm, col2_hbm, out_hbm,
             rowb, colb, hab0, hab1, hab2, hab3, hbb0, hbb1, hbb2, hbb3,
             ob0, ob1,
             sa0, sa1, sa2, sa3, sb0, sb1, sb2, sb3, so0, so1):
        c = lax.axis_index("c")
        s = lax.axis_index("s")
        w = c * NS + s
        pltpu.sync_copy(row2_hbm.at[w], rowb)
        pltpu.sync_copy(col2_hbm.at[w], colb)
        habs = (hab0, hab1, hab2, hab3)
        hbbs = (hbb0, hbb1, hbb2, hbb3)
        obs = (ob0, ob1)
        sas = (sa0, sa1, sa2, sa3)
        sbs = (sb0, sb1, sb2, sb3)
        sos = (so0, so1)
        for b in range(4):
            pltpu.async_copy(ha_hbm.at[rowb.at[b]], habs[b], sas[b])
            pltpu.async_copy(hb_hbm.at[colb.at[b]], hbbs[b], sbs[b])

        def process(b, ob_b, ci, first_reuse):
            pltpu.make_async_copy(
                ha_hbm.at[rowb.at[ci]], habs[b], sas[b]).wait()
            pltpu.make_async_copy(
                hb_hbm.at[colb.at[ci]], hbbs[b], sbs[b]).wait()

            @pl.when(first_reuse)
            def _():
                pltpu.make_async_copy(
                    obs[ob_b], out_hbm.at[pl.ds(0, CH)], sos[ob_b]).wait()

            def rowloop(i, c2):
                for j in range(8):
                    sl = pl.ds(j * 16, 16)
                    obs[ob_b][i, sl] = habs[b][i, sl] + hbbs[b][i, sl]
                return c2

            lax.fori_loop(0, CH, rowloop, 0)
            pltpu.async_copy(
                obs[ob_b], out_hbm.at[pl.ds(w * ew + ci * CH, CH)],
                sos[ob_b])

            @pl.when(ci + 4 < nchunk)
            def _():
                pltpu.async_copy(ha_hbm.at[rowb.at[ci + 4]], habs[b], sas[b])
                pltpu.async_copy(hb_hbm.at[colb.at[ci + 4]], hbbs[b], sbs[b])

        def body4(m4, cc):
            for b in range(4):
                ci = m4 * 4 + b
                process(b, b % 2, ci, ci >= 2)
            return cc

        lax.fori_loop(0, nchunk // 4, body4, 0)
        for r in range(nchunk % 4):
            ci = (nchunk // 4) * 4 + r
            process(r, r % 2, ci, True)
        for b in range(2):
            pltpu.make_async_copy(
                obs[b], out_hbm.at[pl.ds(0, CH)], sos[b]).wait()

    return body


def _sc_gather(ha, hb, row2, col2, e_sz, ew, nchunk):
    fn = functools.partial(
        pl.kernel,
        out_type=jax.ShapeDtypeStruct((e_sz, 128), F32),
        mesh=_sc_mesh(),
        scratch_types=(
            [pltpu.VMEM((nchunk, CH), jnp.int32)] * 2
            + [pltpu.VMEM((CH, 128), F32)] * 10
            + [pltpu.SemaphoreType.DMA] * 10
        ),
    )(_make_gather_body(ew, nchunk))
    return fn(ha, hb, row2, col2)


def _edge_body(gsum_ref, attr_ref, ew_ref, wc_ref, wd_ref, b1_ref,
               w2_ref, b2_ref, m_ref):
    d = ew_ref[...] * INV_CUTOFF
    t = (gsum_ref[...]
         + jnp.dot(attr_ref[...], wc_ref[...].astype(BF16),
                   preferred_element_type=F32)
         + d * wd_ref[...]
         + b1_ref[...])
    t = _silu(t)
    m_ref[...] = _silu(
        jnp.dot(t, w2_ref[...], preferred_element_type=F32) + b2_ref[...])


def _tc_edge(gsum, edge_attr, ew2, wc, wd, b1, w2, b2, e_sz):
    return pl.pallas_call(
        _edge_body,
        grid=(e_sz // BE,),
        in_specs=[
            pl.BlockSpec((BE, 128), lambda i: (i, 0)),
            pl.BlockSpec((BE, 128), lambda i: (i, 0)),
            pl.BlockSpec((BE, 1), lambda i: (i, 0)),
            pl.BlockSpec((128, 128), lambda i: (0, 0)),
            pl.BlockSpec((1, 128), lambda i: (0, 0)),
            pl.BlockSpec((1, 128), lambda i: (0, 0)),
            pl.BlockSpec((128, 128), lambda i: (0, 0)),
            pl.BlockSpec((1, 128), lambda i: (0, 0)),
        ],
        out_specs=pl.BlockSpec((BE, 128), lambda i: (i, 0)),
        out_shape=jax.ShapeDtypeStruct((e_sz, 128), F32),
    )(gsum, edge_attr, ew2, wc, wd, b1, w2, b2)


def _make_scatter_body(ew, nchunk):
    def body(m_hbm, row2_hbm, out_hbm, rowb, mb0, mb1, mb2, mb3, shag,
             sm0, sm1, sm2, sm3, ss0, ss1, ss2, ss3, sz):
        c = lax.axis_index("c")
        s = lax.axis_index("s")
        w = c * NS + s
        mbs = (mb0, mb1, mb2, mb3)
        sms = (sm0, sm1, sm2, sm3)
        sss = (ss0, ss1, ss2, ss3)

        def zrow(i, cc):
            for j in range(8):
                mb0[i, pl.ds(j * 16, 16)] = jnp.zeros((16,), F32)
            return cc

        lax.fori_loop(0, CH, zrow, 0)
        nz = ROWS_PER_TILE // CH
        for k in range(nz):
            pltpu.async_copy(
                mb0, shag.at[pl.ds(s * ROWS_PER_TILE + k * CH, CH)], sz)
        for k in range(nz):
            pltpu.make_async_copy(
                mb0, shag.at[pl.ds(s * ROWS_PER_TILE, CH)], sz).wait()
        plsc.subcore_barrier()

        pltpu.sync_copy(row2_hbm.at[w], rowb)
        for b in range(2):
            pltpu.async_copy(
                m_hbm.at[pl.ds(w * ew + b * CH, CH)], mbs[b], sms[b])

        def load_wait(b, ci):
            pltpu.make_async_copy(
                m_hbm.at[pl.ds(w * ew + ci * CH, CH)], mbs[b], sms[b]).wait()

        def scat_wait(b):
            pltpu.make_async_copy(
                mbs[b], shag.at[rowb.at[0]], sss[b]).wait()

        def process(k, ci):
            b = k % 4
            b2 = (k + 2) % 4
            load_wait(b, ci)
            pltpu.async_copy(mbs[b], shag.at[rowb.at[ci]], sss[b],
                             add=True)

            @pl.when(ci >= 2)
            def _():
                scat_wait(b2)

            @pl.when(ci + 2 < nchunk)
            def _():
                pltpu.async_copy(
                    m_hbm.at[pl.ds(w * ew + (ci + 2) * CH, CH)],
                    mbs[b2], sms[b2])

        def body4(m4, cc):
            for k in range(4):
                process(k, m4 * 4 + k)
            return cc

        lax.fori_loop(0, nchunk // 4, body4, 0)
        for r in range(nchunk % 4):
            process(r, (nchunk // 4) * 4 + r)
        scat_wait((nchunk - 2) % 4)
        scat_wait((nchunk - 1) % 4)
        plsc.subcore_barrier()
        pltpu.sync_copy(shag.at[pl.ds(s * ROWS_PER_TILE, ROWS_PER_TILE)],
                        out_hbm.at[c, pl.ds(s * ROWS_PER_TILE,
                                            ROWS_PER_TILE)])

    return body


def _sc_scatter(m, row2, ew, nchunk):
    fn = functools.partial(
        pl.kernel,
        out_type=jax.ShapeDtypeStruct((NC, NP, 128), F32),
        mesh=_sc_mesh(),
        scratch_types=(
            [pltpu.VMEM((nchunk, CH), jnp.int32)]
            + [pltpu.VMEM((CH, 128), F32)] * 4
            + [pltpu.VMEM_SHARED((NP, 128), F32)]
            + [pltpu.SemaphoreType.DMA] * 9
        ),
    )(_make_scatter_body(ew, nchunk))
    return fn(m, row2)


def _node_body(h_ref, a0_ref, a1_ref, a2_ref, a3_ref, w1h_ref, w1a_ref,
               b1_ref, w2_ref, b2_ref, wa_ref, wb_ref,
               hn_ref, ha_ref, hb_ref):
    agg = (a0_ref[0] + a1_ref[0]) + (a2_ref[0] + a3_ref[0])
    u = _silu(jnp.dot(h_ref[...], w1h_ref[...], preferred_element_type=F32)
              + jnp.dot(agg, w1a_ref[...], preferred_element_type=F32)
              + b1_ref[...])
    hn = h_ref[...] + jnp.dot(u, w2_ref[...],
                              preferred_element_type=F32) + b2_ref[...]
    hn_ref[...] = hn
    ha_ref[...] = jnp.dot(hn, wa_ref[...], preferred_element_type=F32)
    hb_ref[...] = jnp.dot(hn, wb_ref[...], preferred_element_type=F32)


def _tc_node(h, aggpA, aggpB, w1h, w1a, b1, w2, b2, wa, wb):
    wspec = pl.BlockSpec((128, 128), lambda i: (0, 0))
    bspec = pl.BlockSpec((1, 128), lambda i: (0, 0))
    return pl.pallas_call(
        _node_body,
        grid=(N // BN,),
        in_specs=[
            pl.BlockSpec((BN, 128), lambda i: (i, 0)),
            pl.BlockSpec((1, BN, 128), lambda i: (0, i, 0)),
            pl.BlockSpec((1, BN, 128), lambda i: (1, i, 0)),
            pl.BlockSpec((1, BN, 128), lambda i: (0, i, 0)),
            pl.BlockSpec((1, BN, 128), lambda i: (1, i, 0)),
            wspec, wspec, bspec, wspec, bspec, wspec, wspec,
        ],
        out_specs=[
            pl.BlockSpec((BN, 128), lambda i: (i, 0)),
            pl.BlockSpec((BN, 128), lambda i: (i, 0)),
            pl.BlockSpec((BN, 128), lambda i: (i, 0)),
        ],
        out_shape=[
            jax.ShapeDtypeStruct((N, 128), F32),
            jax.ShapeDtypeStruct((N, 128), F32),
            jax.ShapeDtypeStruct((N, 128), F32),
        ],
    )(h, aggpA, aggpA, aggpB, aggpB, w1h, w1a, b1, w2, b2, wa, wb)


def _pool_body(h_ref, batch_ref, linw_ref, linb_ref, out_ref, sums, cnts):
    i = pl.program_id(0)

    @pl.when(i == 0)
    def _():
        sums[...] = jnp.zeros_like(sums)
        cnts[...] = jnp.zeros_like(cnts)

    io = lax.broadcasted_iota(jnp.int32, (BN, G), 1)
    oh = (io == batch_ref[...]).astype(F32)
    dn = (((0,), (0,)), ((), ()))
    sums[...] += lax.dot_general(oh, h_ref[...], dn,
                                 preferred_element_type=F32)
    cnts[...] += lax.dot_general(oh, jnp.ones((BN, 128), F32), dn,
                                 preferred_element_type=F32)

    @pl.when(i == pl.num_programs(0) - 1)
    def _():
        pooled = sums[...] / jnp.maximum(cnts[...], 1.0)
        out_ref[...] = (jnp.dot(jnp.maximum(pooled, 0.0), linw_ref[...],
                                preferred_element_type=F32) + linb_ref[...])


def _tc_pool(h, batch2, lin_w, lin_b):
    return pl.pallas_call(
        _pool_body,
        grid=(N // BN,),
        in_specs=[
            pl.BlockSpec((BN, 128), lambda i: (i, 0)),
            pl.BlockSpec((BN, 1), lambda i: (i, 0)),
            pl.BlockSpec((128, 128), lambda i: (0, 0)),
            pl.BlockSpec((1, 128), lambda i: (0, 0)),
        ],
        out_specs=pl.BlockSpec((G, 128), lambda i: (0, 0)),
        out_shape=jax.ShapeDtypeStruct((G, 128), F32),
        scratch_shapes=[
            pltpu.VMEM((G, 128), F32),
            pltpu.VMEM((G, 128), F32),
        ],
    )(h, batch2, lin_w, lin_b)


def kernel(x, edge_index, edge_weight, edge_attr, batch, params):
    x2 = x.astype(jnp.int32).reshape(N, 1)
    row = edge_index[0].astype(jnp.int32)
    col = edge_index[1].astype(jnp.int32)
    row3 = row.reshape(NW, NCHUNK, CH)
    col3 = col.reshape(NW, NCHUNK, CH)
    rowA, rowB = row3[:, :NCHUNK_H], row3[:, NCHUNK_H:]
    colA, colB = col3[:, :NCHUNK_H], col3[:, NCHUNK_H:]
    batch2 = batch.astype(jnp.int32).reshape(N, 1)

    attr_bf = edge_attr.astype(BF16).reshape(NW, NHALF, EW_H, 128)
    attrA = attr_bf[:, 0].reshape(E_H, 128)
    attrB = attr_bf[:, 1].reshape(E_H, 128)
    ew4 = edge_weight.astype(F32).reshape(NW, NHALF, EW_H, 1)
    ewA = ew4[:, 0].reshape(E_H, 1)
    ewB = ew4[:, 1].reshape(E_H, 1)

    emb_p = jnp.zeros((128, 128), F32).at[:100].set(params['emb'])
    lays = params['layers']
    wa = [lp['e_w1'][0:H] for lp in lays]
    wb = [lp['e_w1'][H:2 * H] for lp in lays]
    wc = [lp['e_w1'][2 * H:2 * H + 128] for lp in lays]
    wd = [lp['e_w1'][2 * H + 128:2 * H + 129] for lp in lays]
    b1 = [lp['e_b1'].reshape(1, H) for lp in lays]
    w2 = [lp['e_w2'] for lp in lays]
    b2 = [lp['e_b2'].reshape(1, H) for lp in lays]
    w1h = [lp['n_w1'][0:H] for lp in lays]
    w1a = [lp['n_w1'][H:2 * H] for lp in lays]
    nb1 = [lp['n_b1'].reshape(1, H) for lp in lays]
    nw2 = [lp['n_w2'] for lp in lays]
    nb2 = [lp['n_b2'].reshape(1, H) for lp in lays]

    h, ha, hb = _tc_init(x2, emb_p, wa[0], wb[0])
    zero_w = jnp.zeros((H, H), F32)
    for l in range(3):
        gA = _sc_gather(ha, hb, rowA, colA, E_H, EW_H, NCHUNK_H)
        mA = _tc_edge(gA, attrA, ewA, wc[l], wd[l], b1[l], w2[l], b2[l], E_H)
        gB = _sc_gather(ha, hb, rowB, colB, E_H, EW_H, NCHUNK_H)
        mB = _tc_edge(gB, attrB, ewB, wc[l], wd[l], b1[l], w2[l], b2[l], E_H)
        aggpA = _sc_scatter(mA, rowA, EW_H, NCHUNK_H)
        aggpB = _sc_scatter(mB, rowB, EW_H, NCHUNK_H)
        nwa = wa[l + 1] if l + 1 < 3 else zero_w
        nwb = wb[l + 1] if l + 1 < 3 else zero_w
        h, ha, hb = _tc_node(h, aggpA, aggpB, w1h[l], w1a[l], nb1[l],
                             nw2[l], nb2[l], nwa, nwb)
    return _tc_pool(h, batch2, params['lin_w'], params['lin_b'].reshape(1, H))


# speedup vs baseline: 4.2272x; 1.0040x over previous
import functools

import jax
import jax.numpy as jnp
from jax import lax
from jax.experimental import pallas as pl
from jax.experimental.pallas import tpu as pltpu
from jax.experimental.pallas import tpu_sc as plsc

N = 10000
E = 320000
H = 128
G = 64
INV_CUTOFF = 1.0 / 5.0

NC = 2
NS = 16
NW = NC * NS
EW = E // NW
CH = 40
NCHUNK = EW // CH
NHALF = 2
E_H = E // NHALF
EW_H = EW // NHALF
NCHUNK_H = NCHUNK // NHALF
NP = 10240
ROWS_PER_TILE = NP // NS

BN = 2000
BE = 4000
F32 = jnp.float32
BF16 = jnp.bfloat16


def _silu(v):
    return v * jax.nn.sigmoid(v)


def _sc_mesh():
    return plsc.VectorSubcoreMesh(
        core_axis_name="c", subcore_axis_name="s",
        num_cores=NC, num_subcores=NS)


def _init_body(x_ref, emb_ref, wa_ref, wb_ref, h_ref, hab_ref):
    io = lax.broadcasted_iota(jnp.int32, (BN, 128), 1)
    oh = (io == x_ref[...]).astype(F32)
    h = jnp.dot(oh, emb_ref[...], preferred_element_type=F32)
    h_ref[...] = h
    hab_ref[0] = jnp.dot(h, wa_ref[...], preferred_element_type=F32)
    hab_ref[1] = jnp.dot(h, wb_ref[...], preferred_element_type=F32)


def _tc_init(x2, emb_p, wa, wb):
    return pl.pallas_call(
        _init_body,
        grid=(N // BN,),
        in_specs=[
            pl.BlockSpec((BN, 1), lambda i: (i, 0)),
            pl.BlockSpec((128, 128), lambda i: (0, 0)),
            pl.BlockSpec((128, 128), lambda i: (0, 0)),
            pl.BlockSpec((128, 128), lambda i: (0, 0)),
        ],
        out_specs=[
            pl.BlockSpec((BN, 128), lambda i: (i, 0)),
            pl.BlockSpec((2, BN, 128), lambda i: (0, i, 0)),
        ],
        out_shape=[
            jax.ShapeDtypeStruct((N, 128), F32),
            jax.ShapeDtypeStruct((2, N, 128), F32),
        ],
    )(x2, emb_p, wa, wb)


def _make_gather_body(ew, nchunk):
    def body(tab_hbm, idx2_hbm, out_hbm,
             idxb, gb0, gb1, gb2, gb3, ob0, ob1,
             sa0, sa1, sa2, sa3, so0, so1):
        c = lax.axis_index("c")
        s = lax.axis_index("s")
        w = c * NS + s
        pltpu.sync_copy(idx2_hbm.at[w], idxb)
        gbs = (gb0, gb1, gb2, gb3)
        obs = (ob0, ob1)
        sas = (sa0, sa1, sa2, sa3)
        sos = (so0, so1)
        for b in range(4):
            pltpu.async_copy(tab_hbm.at[idxb.at[b]], gbs[b], sas[b])

        def process(b, ob_b, ci, first_reuse):
            pltpu.make_async_copy(
                tab_hbm.at[idxb.at[ci]], gbs[b], sas[b]).wait()

            @pl.when(first_reuse)
            def _():
                pltpu.make_async_copy(
                    obs[ob_b], out_hbm.at[pl.ds(0, CH)], sos[ob_b]).wait()

            def rowloop(i, c2):
                for j in range(8):
                    sl = pl.ds(j * 16, 16)
                    obs[ob_b][i, sl] = gbs[b][i, sl] + gbs[b][i + CH, sl]
                return c2

            lax.fori_loop(0, CH, rowloop, 0)
            pltpu.async_copy(
                obs[ob_b], out_hbm.at[pl.ds(w * ew + ci * CH, CH)],
                sos[ob_b])

            @pl.when(ci + 4 < nchunk)
            def _():
                pltpu.async_copy(tab_hbm.at[idxb.at[ci + 4]], gbs[b], sas[b])

        def body4(m4, cc):
            for b in range(4):
                ci = m4 * 4 + b
                process(b, b % 2, ci, ci >= 2)
            return cc

        lax.fori_loop(0, nchunk // 4, body4, 0)
        for r in range(nchunk % 4):
            ci = (nchunk // 4) * 4 + r
            process(r, r % 2, ci, True)
        for b in range(2):
            pltpu.make_async_copy(
                obs[b], out_hbm.at[pl.ds(0, CH)], sos[b]).wait()

    return body


def _sc_gather(tab, idx2, e_sz, ew, nchunk):
    fn = functools.partial(
        pl.kernel,
        out_type=jax.ShapeDtypeStruct((e_sz, 128), F32),
        mesh=_sc_mesh(),
        scratch_types=(
            [pltpu.VMEM((nchunk, 2 * CH), jnp.int32)]
            + [pltpu.VMEM((2 * CH, 128), F32)] * 4
            + [pltpu.VMEM((CH, 128), F32)] * 2
            + [pltpu.SemaphoreType.DMA] * 6
        ),
    )(_make_gather_body(ew, nchunk))
    return fn(tab, idx2)


def _edge_body(gsum_ref, attr_ref, ew_ref, wc_ref, wd_ref, b1_ref,
               w2_ref, b2_ref, m_ref):
    d = ew_ref[...] * INV_CUTOFF
    t = (gsum_ref[...]
         + jnp.dot(attr_ref[...], wc_ref[...].astype(BF16),
                   preferred_element_type=F32)
         + d * wd_ref[...]
         + b1_ref[...])
    t = _silu(t)
    m_ref[...] = _silu(
        jnp.dot(t, w2_ref[...], preferred_element_type=F32) + b2_ref[...])


def _tc_edge(gsum, edge_attr, ew2, wc, wd, b1, w2, b2, e_sz):
    return pl.pallas_call(
        _edge_body,
        grid=(e_sz // BE,),
        in_specs=[
            pl.BlockSpec((BE, 128), lambda i: (i, 0)),
            pl.BlockSpec((BE, 128), lambda i: (i, 0)),
            pl.BlockSpec((BE, 1), lambda i: (i, 0)),
            pl.BlockSpec((128, 128), lambda i: (0, 0)),
            pl.BlockSpec((1, 128), lambda i: (0, 0)),
            pl.BlockSpec((1, 128), lambda i: (0, 0)),
            pl.BlockSpec((128, 128), lambda i: (0, 0)),
            pl.BlockSpec((1, 128), lambda i: (0, 0)),
        ],
        out_specs=pl.BlockSpec((BE, 128), lambda i: (i, 0)),
        out_shape=jax.ShapeDtypeStruct((e_sz, 128), F32),
    )(gsum, edge_attr, ew2, wc, wd, b1, w2, b2)


def _make_scatter_body(ew, nchunk):
    def body(m_hbm, row2_hbm, out_hbm, rowb, mb0, mb1, mb2, mb3, shag,
             sm0, sm1, sm2, sm3, ss0, ss1, ss2, ss3, sz):
        c = lax.axis_index("c")
        s = lax.axis_index("s")
        w = c * NS + s
        mbs = (mb0, mb1, mb2, mb3)
        sms = (sm0, sm1, sm2, sm3)
        sss = (ss0, ss1, ss2, ss3)

        def zrow(i, cc):
            for j in range(8):
                mb0[i, pl.ds(j * 16, 16)] = jnp.zeros((16,), F32)
            return cc

        lax.fori_loop(0, CH, zrow, 0)
        nz = ROWS_PER_TILE // CH
        for k in range(nz):
            pltpu.async_copy(
                mb0, shag.at[pl.ds(s * ROWS_PER_TILE + k * CH, CH)], sz)
        for k in range(nz):
            pltpu.make_async_copy(
                mb0, shag.at[pl.ds(s * ROWS_PER_TILE, CH)], sz).wait()
        plsc.subcore_barrier()

        pltpu.sync_copy(row2_hbm.at[w], rowb)
        for b in range(2):
            pltpu.async_copy(
                m_hbm.at[pl.ds(w * ew + b * CH, CH)], mbs[b], sms[b])

        def load_wait(b, ci):
            pltpu.make_async_copy(
                m_hbm.at[pl.ds(w * ew + ci * CH, CH)], mbs[b], sms[b]).wait()

        def scat_wait(b):
            pltpu.make_async_copy(
                mbs[b], shag.at[rowb.at[0]], sss[b]).wait()

        def process(k, ci):
            b = k % 4
            b2 = (k + 2) % 4
            load_wait(b, ci)
            pltpu.async_copy(mbs[b], shag.at[rowb.at[ci]], sss[b],
                             add=True)

            @pl.when(ci >= 2)
            def _():
                scat_wait(b2)

            @pl.when(ci + 2 < nchunk)
            def _():
                pltpu.async_copy(
                    m_hbm.at[pl.ds(w * ew + (ci + 2) * CH, CH)],
                    mbs[b2], sms[b2])

        def body4(m4, cc):
            for k in range(4):
                process(k, m4 * 4 + k)
            return cc

        lax.fori_loop(0, nchunk // 4, body4, 0)
        for r in range(nchunk % 4):
            process(r, (nchunk // 4) * 4 + r)
        scat_wait((nchunk - 2) % 4)
        scat_wait((nchunk - 1) % 4)
        plsc.subcore_barrier()
        pltpu.sync_copy(shag.at[pl.ds(s * ROWS_PER_TILE, ROWS_PER_TILE)],
                        out_hbm.at[c, pl.ds(s * ROWS_PER_TILE,
                                            ROWS_PER_TILE)])

    return body


def _sc_scatter(m, row2, ew, nchunk):
    fn = functools.partial(
        pl.kernel,
        out_type=jax.ShapeDtypeStruct((NC, NP, 128), F32),
        mesh=_sc_mesh(),
        scratch_types=(
            [pltpu.VMEM((nchunk, CH), jnp.int32)]
            + [pltpu.VMEM((CH, 128), F32)] * 4
            + [pltpu.VMEM_SHARED((NP, 128), F32)]
            + [pltpu.SemaphoreType.DMA] * 9
        ),
    )(_make_scatter_body(ew, nchunk))
    return fn(m, row2)


def _node_body(h_ref, a0_ref, a1_ref, a2_ref, a3_ref, w1h_ref, w1a_ref,
               b1_ref, w2_ref, b2_ref, wa_ref, wb_ref,
               hn_ref, hab_ref):
    agg = (a0_ref[0] + a1_ref[0]) + (a2_ref[0] + a3_ref[0])
    u = _silu(jnp.dot(h_ref[...], w1h_ref[...], preferred_element_type=F32)
              + jnp.dot(agg, w1a_ref[...], preferred_element_type=F32)
              + b1_ref[...])
    hn = h_ref[...] + jnp.dot(u, w2_ref[...],
                              preferred_element_type=F32) + b2_ref[...]
    hn_ref[...] = hn
    hab_ref[0] = jnp.dot(hn, wa_ref[...], preferred_element_type=F32)
    hab_ref[1] = jnp.dot(hn, wb_ref[...], preferred_element_type=F32)


def _tc_node(h, aggpA, aggpB, w1h, w1a, b1, w2, b2, wa, wb):
    wspec = pl.BlockSpec((128, 128), lambda i: (0, 0))
    bspec = pl.BlockSpec((1, 128), lambda i: (0, 0))
    return pl.pallas_call(
        _node_body,
        grid=(N // BN,),
        in_specs=[
            pl.BlockSpec((BN, 128), lambda i: (i, 0)),
            pl.BlockSpec((1, BN, 128), lambda i: (0, i, 0)),
            pl.BlockSpec((1, BN, 128), lambda i: (1, i, 0)),
            pl.BlockSpec((1, BN, 128), lambda i: (0, i, 0)),
            pl.BlockSpec((1, BN, 128), lambda i: (1, i, 0)),
            wspec, wspec, bspec, wspec, bspec, wspec, wspec,
        ],
        out_specs=[
            pl.BlockSpec((BN, 128), lambda i: (i, 0)),
            pl.BlockSpec((2, BN, 128), lambda i: (0, i, 0)),
        ],
        out_shape=[
            jax.ShapeDtypeStruct((N, 128), F32),
            jax.ShapeDtypeStruct((2, N, 128), F32),
        ],
    )(h, aggpA, aggpA, aggpB, aggpB, w1h, w1a, b1, w2, b2, wa, wb)


def _pool_body(h_ref, batch_ref, linw_ref, linb_ref, out_ref, sums, cnts):
    i = pl.program_id(0)

    @pl.when(i == 0)
    def _():
        sums[...] = jnp.zeros_like(sums)
        cnts[...] = jnp.zeros_like(cnts)

    io = lax.broadcasted_iota(jnp.int32, (BN, G), 1)
    oh = (io == batch_ref[...]).astype(F32)
    dn = (((0,), (0,)), ((), ()))
    sums[...] += lax.dot_general(oh, h_ref[...], dn,
                                 preferred_element_type=F32)
    cnts[...] += lax.dot_general(oh, jnp.ones((BN, 128), F32), dn,
                                 preferred_element_type=F32)

    @pl.when(i == pl.num_programs(0) - 1)
    def _():
        pooled = sums[...] / jnp.maximum(cnts[...], 1.0)
        out_ref[...] = (jnp.dot(jnp.maximum(pooled, 0.0), linw_ref[...],
                                preferred_element_type=F32) + linb_ref[...])


def _tc_pool(h, batch2, lin_w, lin_b):
    return pl.pallas_call(
        _pool_body,
        grid=(N // BN,),
        in_specs=[
            pl.BlockSpec((BN, 128), lambda i: (i, 0)),
            pl.BlockSpec((BN, 1), lambda i: (i, 0)),
            pl.BlockSpec((128, 128), lambda i: (0, 0)),
            pl.BlockSpec((1, 128), lambda i: (0, 0)),
        ],
        out_specs=pl.BlockSpec((G, 128), lambda i: (0, 0)),
        out_shape=jax.ShapeDtypeStruct((G, 128), F32),
        scratch_shapes=[
            pltpu.VMEM((G, 128), F32),
            pltpu.VMEM((G, 128), F32),
        ],
    )(h, batch2, lin_w, lin_b)


def kernel(x, edge_index, edge_weight, edge_attr, batch, params):
    x2 = x.astype(jnp.int32).reshape(N, 1)
    row = edge_index[0].astype(jnp.int32)
    col = edge_index[1].astype(jnp.int32)
    row3 = row.reshape(NW, NCHUNK, CH)
    col3 = col.reshape(NW, NCHUNK, CH)
    idx3 = jnp.concatenate([row3, col3 + N], axis=-1)
    rowA, rowB = row3[:, :NCHUNK_H], row3[:, NCHUNK_H:]
    idxA, idxB = idx3[:, :NCHUNK_H], idx3[:, NCHUNK_H:]
    batch2 = batch.astype(jnp.int32).reshape(N, 1)

    attr_bf = edge_attr.astype(BF16).reshape(NW, NHALF, EW_H, 128)
    attrA = attr_bf[:, 0].reshape(E_H, 128)
    attrB = attr_bf[:, 1].reshape(E_H, 128)
    ew4 = edge_weight.astype(F32).reshape(NW, NHALF, EW_H, 1)
    ewA = ew4[:, 0].reshape(E_H, 1)
    ewB = ew4[:, 1].reshape(E_H, 1)

    emb_p = jnp.zeros((128, 128), F32).at[:100].set(params['emb'])
    lays = params['layers']
    wa = [lp['e_w1'][0:H] for lp in lays]
    wb = [lp['e_w1'][H:2 * H] for lp in lays]
    wc = [lp['e_w1'][2 * H:2 * H + 128] for lp in lays]
    wd = [lp['e_w1'][2 * H + 128:2 * H + 129] for lp in lays]
    b1 = [lp['e_b1'].reshape(1, H) for lp in lays]
    w2 = [lp['e_w2'] for lp in lays]
    b2 = [lp['e_b2'].reshape(1, H) for lp in lays]
    w1h = [lp['n_w1'][0:H] for lp in lays]
    w1a = [lp['n_w1'][H:2 * H] for lp in lays]
    nb1 = [lp['n_b1'].reshape(1, H) for lp in lays]
    nw2 = [lp['n_w2'] for lp in lays]
    nb2 = [lp['n_b2'].reshape(1, H) for lp in lays]

    h, hab = _tc_init(x2, emb_p, wa[0], wb[0])
    zero_w = jnp.zeros((H, H), F32)
    for l in range(3):
        tab = hab.reshape(2 * N, 128)
        gA = _sc_gather(tab, idxA, E_H, EW_H, NCHUNK_H)
        mA = _tc_edge(gA, attrA, ewA, wc[l], wd[l], b1[l], w2[l], b2[l], E_H)
        gB = _sc_gather(tab, idxB, E_H, EW_H, NCHUNK_H)
        mB = _tc_edge(gB, attrB, ewB, wc[l], wd[l], b1[l], w2[l], b2[l], E_H)
        aggpA = _sc_scatter(mA, rowA, EW_H, NCHUNK_H)
        aggpB = _sc_scatter(mB, rowB, EW_H, NCHUNK_H)
        nwa = wa[l + 1] if l + 1 < 3 else zero_w
        nwb = wb[l + 1] if l + 1 < 3 else zero_w
        h, hab = _tc_node(h, aggpA, aggpB, w1h[l], w1a[l], nb1[l],
                          nw2[l], nb2[l], nwa, nwb)
    return _tc_pool(h, batch2, params['lin_w'], params['lin_b'].reshape(1, H))
